# baseline (speedup 1.0000x reference)
import functools

import jax
import jax.numpy as jnp
from jax import lax
from jax.experimental import pallas as pl
from jax.experimental.pallas import tpu as pltpu
from jax.experimental.pallas import tpu_sc as plsc

_B = 64
_N = 100000
_RG = 8


_GUMBEL_CACHE = []


def _gumbel_const():
    if not _GUMBEL_CACHE:
        g = jax.random.gumbel(jax.random.key(42), (_B, _N), jnp.float32)
        _GUMBEL_CACHE.append(jax.block_until_ready(g))
    return _GUMBEL_CACHE[0]


_P = 4
_W = _N // _P


def _tc_body(*refs):
    lg_refs = refs[:_P]
    gm_refs = refs[_P:2 * _P]
    flat_idx_ref, logp_ref = refs[2 * _P], refs[2 * _P + 1]

    ii = lax.broadcasted_iota(jnp.int32, (_RG, _W), 1)
    m = s = xm = fi = ch = None
    for p in range(_P):
        lg = lg_refs[p][:, 0, 0, :]
        x = lg + gm_refs[p][:, 0, 0, :]
        m_p = jnp.max(lg, axis=1, keepdims=True)
        s_p = jnp.sum(jnp.exp(lg - m_p), axis=1, keepdims=True)
        xm_p = jnp.max(x, axis=1, keepdims=True)
        fi_p = jnp.min(jnp.where(x == xm_p, ii, _W), axis=1, keepdims=True)
        ch_p = jnp.sum(jnp.where(ii == fi_p, lg, 0.0), axis=1, keepdims=True)
        fi_p = fi_p + p * _W
        if p == 0:
            m, s, xm, fi, ch = m_p, s_p, xm_p, fi_p, ch_p
        else:
            better = xm_p > xm
            fi = jnp.where(better, fi_p, fi)
            ch = jnp.where(better, ch_p, ch)
            xm = jnp.maximum(xm, xm_p)
            new_m = jnp.maximum(m, m_p)
            s = s * jnp.exp(m - new_m) + s_p * jnp.exp(m_p - new_m)
            m = new_m
    lse = m + jnp.log(s)
    rows = pl.program_id(0) * _RG + lax.broadcasted_iota(jnp.int32, (_RG, 1), 0)
    flat_idx_ref[...] = jnp.broadcast_to(rows * _N + fi, (_RG, 128))
    logp_ref[...] = jnp.broadcast_to(ch - lse, (_RG, 128))


def _part_spec(p):
    return pl.BlockSpec((_RG, 1, 1, _W), lambda i, p=p: (i, p, 0, 0))


def _tc_stats(logits, gumbel, interpret=False):
    lg3 = logits.reshape(_B, _P, 1, _W)
    gm3 = gumbel.reshape(_B, _P, 1, _W)
    return pl.pallas_call(
        _tc_body,
        interpret=interpret,
        grid=(_B // _RG,),
        in_specs=[_part_spec(p) for p in range(_P)] * 2,
        out_specs=[
            pl.BlockSpec((_RG, 128), lambda i: (i, 0)),
            pl.BlockSpec((_RG, 128), lambda i: (i, 0)),
        ],
        out_shape=[
            jax.ShapeDtypeStruct((_B, 128), jnp.int32),
            jax.ShapeDtypeStruct((_B, 128), jnp.float32),
        ],
    )(*([lg3] * _P + [gm3] * _P))


def _sc_gather_body(flat_hbm, idx_hbm, out_hbm, idx_v, vals_v, sem):
    wid = lax.axis_index("s") * 2 + lax.axis_index("c")

    @pl.when(wid == 0)
    def _():
        pltpu.sync_copy(idx_hbm, idx_v)
        pltpu.async_copy(flat_hbm.at[idx_v], vals_v, sem).wait()
        pltpu.sync_copy(vals_v, out_hbm)


@functools.cache
def _sc_gather():
    return pl.kernel(
        _sc_gather_body,
        out_type=jax.ShapeDtypeStruct((_B,), jnp.int32),
        mesh=plsc.VectorSubcoreMesh(core_axis_name="c", subcore_axis_name="s"),
        scratch_types=[
            pltpu.VMEM((_B,), jnp.int32),
            pltpu.VMEM((_B,), jnp.int32),
            pltpu.SemaphoreType.DMA,
        ],
    )


def kernel(all_logits_list, all_action_inds_list):
    gumbel = _gumbel_const()
    flat_idx, logp = _tc_stats(all_logits_list, gumbel)
    actions = _sc_gather()(all_action_inds_list.reshape(-1), flat_idx[:, 0])
    return actions, logp[:, 0]


# speedup vs baseline: 11.5446x; 11.5446x over previous
import functools

import jax
import jax.numpy as jnp
from jax import lax
from jax.experimental import pallas as pl
from jax.experimental.pallas import tpu as pltpu
from jax.experimental.pallas import tpu_sc as plsc

_B = 64
_N = 100000
_RG = 8


_GUMBEL_CACHE = []


def _gumbel_const():
    if not _GUMBEL_CACHE:
        with jax.ensure_compile_time_eval():
            g = jax.random.gumbel(jax.random.key(42), (_B, _N), jnp.float32)
        _GUMBEL_CACHE.append(g)
    return _GUMBEL_CACHE[0]


def _tc_body(logits_ref, gum_ref, flat_idx_ref, logp_ref):
    lg = logits_ref[...]
    x = lg + gum_ref[...]
    m = jnp.max(lg, axis=1, keepdims=True)
    s = jnp.sum(jnp.exp(lg - m), axis=1, keepdims=True)
    lse = m + jnp.log(s)
    xm = jnp.max(x, axis=1, keepdims=True)
    ii = lax.broadcasted_iota(jnp.int32, (_RG, _N), 1)
    fi = jnp.min(jnp.where(x == xm, ii, _N), axis=1, keepdims=True)
    ch = jnp.sum(jnp.where(ii == fi, lg, 0.0), axis=1, keepdims=True)
    rows = pl.program_id(0) * _RG + lax.broadcasted_iota(jnp.int32, (_RG, 1), 0)
    flat_idx_ref[...] = jnp.broadcast_to(rows * _N + fi, (_RG, 128))
    logp_ref[...] = jnp.broadcast_to(ch - lse, (_RG, 128))


def _tc_stats(logits, gumbel, interpret=False):
    return pl.pallas_call(
        _tc_body,
        interpret=interpret,
        grid=(_B // _RG,),
        in_specs=[
            pl.BlockSpec((_RG, _N), lambda i: (i, 0)),
            pl.BlockSpec((_RG, _N), lambda i: (i, 0)),
        ],
        out_specs=[
            pl.BlockSpec((_RG, 128), lambda i: (i, 0)),
            pl.BlockSpec((_RG, 128), lambda i: (i, 0)),
        ],
        out_shape=[
            jax.ShapeDtypeStruct((_B, 128), jnp.int32),
            jax.ShapeDtypeStruct((_B, 128), jnp.float32),
        ],
    )(logits, gumbel)


def _sc_gather_body(flat_hbm, idx_hbm, out_hbm, idx_v, vals_v, sem):
    wid = lax.axis_index("s") * 2 + lax.axis_index("c")

    @pl.when(wid == 0)
    def _():
        pltpu.sync_copy(idx_hbm, idx_v)
        pltpu.async_copy(flat_hbm.at[idx_v], vals_v, sem).wait()
        pltpu.sync_copy(vals_v, out_hbm)


@functools.cache
def _sc_gather():
    return pl.kernel(
        _sc_gather_body,
        out_type=jax.ShapeDtypeStruct((_B,), jnp.int32),
        mesh=plsc.VectorSubcoreMesh(core_axis_name="c", subcore_axis_name="s"),
        scratch_types=[
            pltpu.VMEM((_B,), jnp.int32),
            pltpu.VMEM((_B,), jnp.int32),
            pltpu.SemaphoreType.DMA,
        ],
    )


def kernel(all_logits_list, all_action_inds_list):
    gumbel = _gumbel_const()
    flat_idx, logp = _tc_stats(all_logits_list, gumbel)
    actions = _sc_gather()(all_action_inds_list.reshape(-1), flat_idx[:, 0])
    return actions, logp[:, 0]
